# baseline (speedup 1.0000x reference)
import jax
import jax.numpy as jnp
import numpy as np
from jax import lax
from jax.experimental import pallas as pl

_DEPTH = 1000
_L = 20
_BS = 128


def _body(x_ref, s_ref, o_ref):
    x = x_ref[...]
    l = lax.broadcasted_iota(jnp.int32, x.shape, 1)
    q = (x * _L + l).astype(jnp.float32)
    xt = lax.dot_general(q, s_ref[...], (((1,), (0,)), ((), ())),
                         precision=lax.Precision.HIGHEST,
                         preferred_element_type=jnp.float32)
    p = lax.broadcasted_iota(jnp.int32, xt.shape, 1).astype(jnp.float32)
    o_ref[...] = (jnp.abs(xt - p) < 0.5).astype(jnp.float32)


def kernel(X_in, ones):
    del ones
    B, L = X_in.shape
    P = _DEPTH * L
    sel = (np.arange(P, dtype=np.int32) % L)[None, :] == np.arange(L, dtype=np.int32)[:, None]
    S = jnp.asarray(sel, dtype=jnp.float32)
    out_flat = pl.pallas_call(
        _body,
        grid=(B // _BS,),
        in_specs=[
            pl.BlockSpec((_BS, L), lambda i: (i, 0)),
            pl.BlockSpec((L, P), lambda i: (0, 0)),
        ],
        out_specs=pl.BlockSpec((_BS, P), lambda i: (i, 0)),
        out_shape=jax.ShapeDtypeStruct((B, P), jnp.float32),
    )(X_in, S)
    return out_flat.reshape(B, _DEPTH, L)


# speedup vs baseline: 1.1623x; 1.1623x over previous
import jax
import jax.numpy as jnp
from jax import lax
from jax.experimental import pallas as pl
from jax.experimental.pallas import tpu as pltpu

_DEPTH = 1000
_L = 20
_BS = 8
_NBUF = 8


def _body(x_ref, o_hbm, buf, sems):
    i = pl.program_id(0)
    n = pl.num_programs(0)
    slot = lax.rem(i, _NBUF)

    @pl.when(i >= _NBUF)
    def _wait_prev():
        prev = i - _NBUF
        pltpu.make_async_copy(
            buf.at[slot], o_hbm.at[pl.ds(prev * _BS, _BS)], sems.at[slot]
        ).wait()

    x = x_ref[...]
    d = lax.broadcasted_iota(jnp.int32, (_BS, _DEPTH, _L), 1)
    buf[slot] = (d == x[:, None, :]).astype(jnp.float32)
    pltpu.make_async_copy(
        buf.at[slot], o_hbm.at[pl.ds(i * _BS, _BS)], sems.at[slot]
    ).start()

    @pl.when(i == n - 1)
    def _drain():
        for k in range(_NBUF):
            step = n - _NBUF + k
            pltpu.make_async_copy(
                buf.at[k], o_hbm.at[pl.ds(step * _BS, _BS)], sems.at[k]
            ).wait()


def kernel(X_in, ones):
    del ones
    B, L = X_in.shape
    return pl.pallas_call(
        _body,
        grid=(B // _BS,),
        in_specs=[pl.BlockSpec((_BS, L), lambda i: (i, 0))],
        out_specs=pl.BlockSpec(memory_space=pl.ANY),
        out_shape=jax.ShapeDtypeStruct((B, _DEPTH, L), jnp.float32),
        scratch_shapes=[
            pltpu.VMEM((_NBUF, _BS, _DEPTH, _L), jnp.float32),
            pltpu.SemaphoreType.DMA((_NBUF,)),
        ],
    )(X_in)


# speedup vs baseline: 20.7783x; 17.8769x over previous
import jax
import jax.numpy as jnp
from jax import lax
from jax.experimental import pallas as pl

_DEPTH = 1000
_L = 20
_BS = 128


def _body(x_ref, o_ref):
    xt = x_ref[...]
    d = lax.broadcasted_iota(jnp.int32, (_L, _DEPTH, _BS), 1)
    o_ref[...] = (xt[:, None, :] == d).astype(jnp.float32)


def kernel(X_in, ones):
    del ones
    B, L = X_in.shape
    XT = X_in.T
    out_phys = pl.pallas_call(
        _body,
        grid=(B // _BS,),
        in_specs=[pl.BlockSpec((L, _BS), lambda i: (0, i))],
        out_specs=pl.BlockSpec((_L, _DEPTH, _BS), lambda i: (0, 0, i)),
        out_shape=jax.ShapeDtypeStruct((L, _DEPTH, B), jnp.float32),
    )(XT)
    return jnp.transpose(out_phys, (2, 1, 0))
